# Initial kernel scaffold; baseline (speedup 1.0000x reference)
import jax
import jax.numpy as jnp
from jax.experimental import pallas as pl


def kernel(x, edge_index, weight, root, bias):
    raise NotImplementedError("write your pallas kernel here")


# speedup vs baseline: 5.5332x; 5.5332x over previous
import functools

import jax
import jax.numpy as jnp
from jax import lax
from jax.experimental import pallas as pl
from jax.experimental.pallas import tpu as pltpu
from jax.experimental.pallas import tpu_sc as plsc

N_NODES_C = 10000
N_EDGES_C = 320000
CH = 128

_NC = 2
_NS = 16
_NW = _NC * _NS
_EDGES_PER_TILE = N_EDGES_C // _NW
_B = 80
_NBATCH = _EDGES_PER_TILE // _B


def _sc_body(x_hbm, row_hbm, col_hbm, zacc_hbm, zdeg_hbm,
             acc_out, deg_out,
             acc_sh, deg_sh, idx_r, idx_c, vals, ones, wb, wbd, sem):
    c = lax.axis_index("c")
    s = lax.axis_index("s")
    wid = c * _NS + s

    for i in range(_B // 16):
        ones[pl.ds(i * 16, 16)] = jnp.ones((16,), jnp.float32)

    @pl.when(s == 0)
    def _():
        pltpu.sync_copy(zacc_hbm, acc_sh)
        pltpu.sync_copy(zdeg_hbm, deg_sh)

    plsc.subcore_barrier()

    base = wid * _EDGES_PER_TILE

    def step(j, carry):
        off = pl.multiple_of(base + j * _B, 8)
        pltpu.sync_copy(row_hbm.at[pl.ds(off, _B)], idx_r)
        pltpu.sync_copy(col_hbm.at[pl.ds(off, _B)], idx_c)
        pltpu.async_copy(x_hbm.at[idx_c], vals, sem).wait()
        pltpu.sync_copy(vals, acc_sh.at[idx_r], add=True)
        pltpu.sync_copy(ones, deg_sh.at[idx_r], add=True)
        return carry

    lax.fori_loop(0, _NBATCH, step, 0)

    plsc.subcore_barrier()

    @pl.when(s < 10)
    def _():
        for k in range(5):
            r0 = pl.multiple_of(s * 1000 + k * 200, 8)
            pltpu.sync_copy(acc_sh.at[pl.ds(r0, 200)], wb)
            pltpu.sync_copy(wb, acc_out.at[c, pl.ds(r0, 200)])
        r0 = pl.multiple_of(s * 1000, 8)
        d0 = pl.multiple_of(c * N_NODES_C + s * 1000, 8)
        pltpu.sync_copy(deg_sh.at[pl.ds(r0, 1000)], wbd)
        pltpu.sync_copy(wbd, deg_out.at[pl.ds(d0, 1000)])


@jax.jit
def _sc_scatter(x, row, col):
    zacc = jnp.zeros((N_NODES_C, CH), jnp.float32)
    zdeg = jnp.zeros((N_NODES_C,), jnp.float32)
    mesh = plsc.VectorSubcoreMesh(core_axis_name="c", subcore_axis_name="s")
    f = pl.kernel(
        _sc_body,
        out_type=[
            jax.ShapeDtypeStruct((_NC, N_NODES_C, CH), jnp.float32),
            jax.ShapeDtypeStruct((_NC * N_NODES_C,), jnp.float32),
        ],
        mesh=mesh,
        scratch_types=[
            pltpu.VMEM_SHARED((N_NODES_C, CH), jnp.float32),
            pltpu.VMEM_SHARED((N_NODES_C,), jnp.float32),
            pltpu.VMEM((_B,), jnp.int32),
            pltpu.VMEM((_B,), jnp.int32),
            pltpu.VMEM((_B, CH), jnp.float32),
            pltpu.VMEM((_B,), jnp.float32),
            pltpu.VMEM((200, CH), jnp.float32),
            pltpu.VMEM((1000,), jnp.float32),
            pltpu.SemaphoreType.DMA,
        ],
    )
    return f(x, row, col, zacc, zdeg)


def _tc_body(acc_ref, deg_ref, x_ref, w_ref, root_ref, bias_ref, o_ref):
    deg = jnp.maximum(deg_ref[0] + deg_ref[1], 1.0)
    agg = (acc_ref[0] + acc_ref[1]) / deg
    o_ref[...] = (
        jnp.dot(agg, w_ref[...], preferred_element_type=jnp.float32)
        + jnp.dot(x_ref[...], root_ref[...], preferred_element_type=jnp.float32)
        + bias_ref[...]
    )


@jax.jit
def _tc_combine(acc2, deg3, x, weight, root, bias2):
    R = 1000
    grid = (N_NODES_C // R,)
    return pl.pallas_call(
        _tc_body,
        grid=grid,
        in_specs=[
            pl.BlockSpec((_NC, R, CH), lambda i: (0, i, 0)),
            pl.BlockSpec((_NC, R, 1), lambda i: (0, i, 0)),
            pl.BlockSpec((R, CH), lambda i: (i, 0)),
            pl.BlockSpec((CH, CH), lambda i: (0, 0)),
            pl.BlockSpec((CH, CH), lambda i: (0, 0)),
            pl.BlockSpec((1, CH), lambda i: (0, 0)),
        ],
        out_specs=pl.BlockSpec((R, CH), lambda i: (i, 0)),
        out_shape=jax.ShapeDtypeStruct((N_NODES_C, CH), jnp.float32),
    )(acc2, deg3, x, weight, root, bias2)


def kernel(x, edge_index, weight, root, bias):
    row = edge_index[0].astype(jnp.int32)
    col = edge_index[1].astype(jnp.int32)
    acc2, deg2 = _sc_scatter(x, row, col)
    deg3 = deg2.reshape(_NC, N_NODES_C, 1)
    return _tc_combine(acc2, deg3, x, weight, root, bias[None, :])


# speedup vs baseline: 8.3106x; 1.5019x over previous
import functools

import jax
import jax.numpy as jnp
from jax import lax
from jax.experimental import pallas as pl
from jax.experimental.pallas import tpu as pltpu
from jax.experimental.pallas import tpu_sc as plsc

N_NODES_C = 10000
N_EDGES_C = 320000
CH = 128

_NC = 2
_NS = 16
_NW = _NC * _NS
_EDGES_PER_TILE = N_EDGES_C // _NW
_B = 80
_NBATCH = _EDGES_PER_TILE // _B


def _sc_body(x_hbm, row_hbm, col_hbm, zacc_hbm, zdeg_hbm,
             acc_out, deg_out,
             acc_sh, deg_sh, r0b, c0b, r1b, c1b, v0, v1, ones, wb, wbd,
             sem0, sem1):
    c = lax.axis_index("c")
    s = lax.axis_index("s")
    wid = c * _NS + s

    for i in range(_B // 16):
        ones[pl.ds(i * 16, 16)] = jnp.ones((16,), jnp.float32)

    @pl.when(s == 0)
    def _():
        pltpu.sync_copy(zacc_hbm, acc_sh)
        pltpu.sync_copy(zdeg_hbm, deg_sh)

    plsc.subcore_barrier()

    base = wid * _EDGES_PER_TILE

    def load_idx(off, rb, cb):
        pltpu.sync_copy(row_hbm.at[pl.ds(off, _B)], rb)
        pltpu.sync_copy(col_hbm.at[pl.ds(off, _B)], cb)

    def flush(rb, cb, v, sem):
        pltpu.make_async_copy(x_hbm.at[cb], v, sem).wait()
        pltpu.sync_copy(v, acc_sh.at[rb], add=True)
        pltpu.sync_copy(ones, deg_sh.at[rb], add=True)

    load_idx(pl.multiple_of(base, 8), r0b, c0b)
    pltpu.async_copy(x_hbm.at[c0b], v0, sem0)

    def step(k, carry):
        j0 = 2 * k
        off1 = pl.multiple_of(base + (j0 + 1) * _B, 8)
        load_idx(off1, r1b, c1b)
        pltpu.async_copy(x_hbm.at[c1b], v1, sem1)
        flush(r0b, c0b, v0, sem0)
        off2 = pl.multiple_of(base + (j0 + 2) * _B, 8)
        load_idx(off2, r0b, c0b)
        pltpu.async_copy(x_hbm.at[c0b], v0, sem0)
        flush(r1b, c1b, v1, sem1)
        return carry

    lax.fori_loop(0, (_NBATCH - 1) // 2, step, 0)
    flush(r0b, c0b, v0, sem0)

    plsc.subcore_barrier()

    @pl.when(s < 10)
    def _():
        for k in range(5):
            r0 = pl.multiple_of(s * 1000 + k * 200, 8)
            pltpu.sync_copy(acc_sh.at[pl.ds(r0, 200)], wb)
            pltpu.sync_copy(wb, acc_out.at[c, pl.ds(r0, 200)])
        r0 = pl.multiple_of(s * 1000, 8)
        d0 = pl.multiple_of(c * N_NODES_C + s * 1000, 8)
        pltpu.sync_copy(deg_sh.at[pl.ds(r0, 1000)], wbd)
        pltpu.sync_copy(wbd, deg_out.at[pl.ds(d0, 1000)])


@jax.jit
def _sc_scatter(x, row, col):
    zacc = jnp.zeros((N_NODES_C, CH), jnp.float32)
    zdeg = jnp.zeros((N_NODES_C,), jnp.float32)
    mesh = plsc.VectorSubcoreMesh(core_axis_name="c", subcore_axis_name="s")
    f = pl.kernel(
        _sc_body,
        out_type=[
            jax.ShapeDtypeStruct((_NC, N_NODES_C, CH), jnp.float32),
            jax.ShapeDtypeStruct((_NC * N_NODES_C,), jnp.float32),
        ],
        mesh=mesh,
        scratch_types=[
            pltpu.VMEM_SHARED((N_NODES_C, CH), jnp.float32),
            pltpu.VMEM_SHARED((N_NODES_C,), jnp.float32),
            pltpu.VMEM((_B,), jnp.int32),
            pltpu.VMEM((_B,), jnp.int32),
            pltpu.VMEM((_B,), jnp.int32),
            pltpu.VMEM((_B,), jnp.int32),
            pltpu.VMEM((_B, CH), jnp.float32),
            pltpu.VMEM((_B, CH), jnp.float32),
            pltpu.VMEM((_B,), jnp.float32),
            pltpu.VMEM((200, CH), jnp.float32),
            pltpu.VMEM((1000,), jnp.float32),
            pltpu.SemaphoreType.DMA,
            pltpu.SemaphoreType.DMA,
        ],
    )
    return f(x, row, col, zacc, zdeg)


def _tc_body(acc_ref, deg_ref, x_ref, w_ref, root_ref, bias_ref, o_ref):
    deg = jnp.maximum(deg_ref[0] + deg_ref[1], 1.0)
    agg = (acc_ref[0] + acc_ref[1]) / deg
    o_ref[...] = (
        jnp.dot(agg, w_ref[...], preferred_element_type=jnp.float32)
        + jnp.dot(x_ref[...], root_ref[...], preferred_element_type=jnp.float32)
        + bias_ref[...]
    )


@jax.jit
def _tc_combine(acc2, deg3, x, weight, root, bias2):
    R = 1000
    grid = (N_NODES_C // R,)
    return pl.pallas_call(
        _tc_body,
        grid=grid,
        in_specs=[
            pl.BlockSpec((_NC, R, CH), lambda i: (0, i, 0)),
            pl.BlockSpec((_NC, R, 1), lambda i: (0, i, 0)),
            pl.BlockSpec((R, CH), lambda i: (i, 0)),
            pl.BlockSpec((CH, CH), lambda i: (0, 0)),
            pl.BlockSpec((CH, CH), lambda i: (0, 0)),
            pl.BlockSpec((1, CH), lambda i: (0, 0)),
        ],
        out_specs=pl.BlockSpec((R, CH), lambda i: (i, 0)),
        out_shape=jax.ShapeDtypeStruct((N_NODES_C, CH), jnp.float32),
    )(acc2, deg3, x, weight, root, bias2)


def kernel(x, edge_index, weight, root, bias):
    row = edge_index[0].astype(jnp.int32)
    col = edge_index[1].astype(jnp.int32)
    acc2, deg2 = _sc_scatter(x, row, col)
    deg3 = deg2.reshape(_NC, N_NODES_C, 1)
    return _tc_combine(acc2, deg3, x, weight, root, bias[None, :])
